# Initial kernel scaffold; baseline (speedup 1.0000x reference)
import jax
import jax.numpy as jnp
from jax.experimental import pallas as pl


def kernel(tokens, weight):
    raise NotImplementedError("write your pallas kernel here")


# speedup vs baseline: 1.8395x; 1.8395x over previous
import functools

import jax
import jax.numpy as jnp
from jax import lax
from jax.experimental import pallas as pl
from jax.experimental.pallas import tpu as pltpu
from jax.experimental.pallas import tpu_sc as plsc

_NC = 2
_NS = 16
_NW = _NC * _NS
_CHUNK = 128


def _make_gather(n_rows: int, embed: int, rows_per_w: int):
    mesh = plsc.VectorSubcoreMesh(
        core_axis_name="c", subcore_axis_name="s",
        num_cores=_NC, num_subcores=_NS,
    )

    @functools.partial(
        pl.kernel,
        mesh=mesh,
        out_type=jax.ShapeDtypeStruct((n_rows * _CHUNK, embed), jnp.float32),
        scratch_types=[
            pltpu.VMEM((rows_per_w, _CHUNK), jnp.int32),
            pltpu.VMEM((2, _CHUNK, embed), jnp.float32),
            pltpu.SemaphoreType.DMA,
            pltpu.SemaphoreType.DMA,
        ],
        compiler_params=pltpu.CompilerParams(use_tc_tiling_on_sc=False),
    )
    def gather_kernel(tok_hbm, w_hbm, out_hbm, idx_v, rows_v, gsem, ssem):
        wid = lax.axis_index("s") * _NC + lax.axis_index("c")
        rbase = wid * rows_per_w
        pltpu.sync_copy(tok_hbm.at[pl.ds(rbase, rows_per_w)], idx_v)

        pltpu.async_copy(w_hbm.at[idx_v.at[0]], rows_v.at[0], gsem)

        def step(j, _):
            slot = lax.rem(j, 2)
            nxt = lax.rem(j + 1, 2)

            @pl.when(j >= 1)
            def _():
                pltpu.make_async_copy(
                    rows_v.at[nxt],
                    out_hbm.at[pl.ds((rbase + j - 1) * _CHUNK, _CHUNK)],
                    ssem,
                ).wait()

            @pl.when(j + 1 < rows_per_w)
            def _():
                pltpu.async_copy(w_hbm.at[idx_v.at[j + 1]], rows_v.at[nxt], gsem)

            pltpu.make_async_copy(w_hbm.at[idx_v.at[j]], rows_v.at[slot], gsem).wait()

            pltpu.async_copy(
                rows_v.at[slot],
                out_hbm.at[pl.ds((rbase + j) * _CHUNK, _CHUNK)],
                ssem,
            )
            return 0

        lax.fori_loop(0, rows_per_w, step, 0, unroll=False)

        last = lax.rem(rows_per_w - 1, 2)
        pltpu.make_async_copy(
            rows_v.at[last],
            out_hbm.at[pl.ds((rbase + rows_per_w - 1) * _CHUNK, _CHUNK)],
            ssem,
        ).wait()

    return gather_kernel


def kernel(tokens, weight):
    b, l = tokens.shape
    vocab, embed = weight.shape
    total = b * l
    assert total % (_NW * _CHUNK) == 0
    n_rows = total // _CHUNK
    rows_per_w = n_rows // _NW

    tok2d = tokens.reshape(n_rows, _CHUNK).astype(jnp.int32)
    out = _make_gather(n_rows, embed, rows_per_w)(tok2d, weight)
    return out.reshape(b, l, embed)


# speedup vs baseline: 1.8780x; 1.0210x over previous
import functools

import jax
import jax.numpy as jnp
from jax import lax
from jax.experimental import pallas as pl
from jax.experimental.pallas import tpu as pltpu
from jax.experimental.pallas import tpu_sc as plsc

_NC = 2
_NS = 16
_NW = _NC * _NS
_CHUNK = 128


_NBUF = 6
_SLOT_CHUNKS = 2
_LOOKAHEAD = 3
_SLOT_ROWS = _SLOT_CHUNKS * _CHUNK


def _make_gather(n_rows: int, embed: int, rows_per_w: int):
    mesh = plsc.VectorSubcoreMesh(
        core_axis_name="c", subcore_axis_name="s",
        num_cores=_NC, num_subcores=_NS,
    )
    n_iters = rows_per_w // _SLOT_CHUNKS

    @functools.partial(
        pl.kernel,
        mesh=mesh,
        out_type=jax.ShapeDtypeStruct((n_rows * _CHUNK, embed), jnp.float32),
        scratch_types=[
            pltpu.VMEM((rows_per_w, _CHUNK), jnp.int32),
            pltpu.VMEM((_NBUF, _SLOT_ROWS, embed), jnp.float32),
            pltpu.SemaphoreType.DMA,
            pltpu.SemaphoreType.DMA,
        ],
        compiler_params=pltpu.CompilerParams(use_tc_tiling_on_sc=False),
    )
    def gather_kernel(tok_hbm, w_hbm, out_hbm, idx_v, rows_v, gsem, ssem):
        wid = lax.axis_index("s") * _NC + lax.axis_index("c")
        rbase = wid * rows_per_w
        pltpu.sync_copy(tok_hbm.at[pl.ds(rbase, rows_per_w)], idx_v)

        def fire_gathers(k):
            slot = lax.rem(k, _NBUF)
            for c in range(_SLOT_CHUNKS):
                pltpu.async_copy(
                    w_hbm.at[idx_v.at[k * _SLOT_CHUNKS + c]],
                    rows_v.at[slot, pl.ds(c * _CHUNK, _CHUNK)],
                    gsem,
                )

        def wait_gathers(k):
            slot = lax.rem(k, _NBUF)
            for c in range(_SLOT_CHUNKS):
                pltpu.make_async_copy(
                    w_hbm.at[idx_v.at[k * _SLOT_CHUNKS + c]],
                    rows_v.at[slot, pl.ds(c * _CHUNK, _CHUNK)],
                    gsem,
                ).wait()

        def scatter_desc(k):
            slot = lax.rem(k, _NBUF)
            return pltpu.make_async_copy(
                rows_v.at[slot],
                out_hbm.at[pl.ds((rbase + k * _SLOT_CHUNKS) * _CHUNK, _SLOT_ROWS)],
                ssem,
            )

        for k in range(_LOOKAHEAD):
            fire_gathers(k)

        def step(j, _):
            wait_gathers(j)
            scatter_desc(j).start()

            @pl.when(j >= _LOOKAHEAD)
            def _():
                scatter_desc(j - _LOOKAHEAD).wait()

            @pl.when(j + _LOOKAHEAD < n_iters)
            def _():
                fire_gathers(j + _LOOKAHEAD)

            return 0

        lax.fori_loop(0, n_iters, step, 0, unroll=False)

        for k in range(n_iters - _LOOKAHEAD, n_iters):
            scatter_desc(k).wait()

    return gather_kernel


def kernel(tokens, weight):
    b, l = tokens.shape
    vocab, embed = weight.shape
    total = b * l
    assert total % (_NW * _CHUNK) == 0
    n_rows = total // _CHUNK
    rows_per_w = n_rows // _NW

    tok2d = tokens.reshape(n_rows, _CHUNK).astype(jnp.int32)
    out = _make_gather(n_rows, embed, rows_per_w)(tok2d, weight)
    return out.reshape(b, l, embed)
